# Initial kernel scaffold; baseline (speedup 1.0000x reference)
import jax
import jax.numpy as jnp
from jax.experimental import pallas as pl


def kernel(x, batch, W1_0, b1_0, W2_0, b2_0, W1_1, b1_1, W2_1, b2_1, W1_2, b1_2, W2_2, b2_2, W1_3, b1_3, W2_3, b2_3):
    raise NotImplementedError("write your pallas kernel here")


# speedup vs baseline: 20.9590x; 20.9590x over previous
import functools

import jax
import jax.numpy as jnp
from jax import lax
from jax.experimental import pallas as pl
from jax.experimental.pallas import tpu as pltpu

NSEG = 512
D = 128
NHEADS = 4
BLK = 1024


def _fused_body(batch_ref, x_ref, w1_ref, b1_ref, w2_ref, b2_ref,
                out_ref, g_scr, se_scr, k_scr):
    p = pl.program_id(0)
    i = pl.program_id(1)
    nblk = pl.num_programs(1)

    @pl.when(p == 0)
    def _phase0():
        xb = x_ref[...]
        h = jnp.maximum(
            jnp.dot(xb, w1_ref[...], preferred_element_type=jnp.float32)
            + b1_ref[...], 0.0)
        g = (jnp.dot(h, w2_ref[...], preferred_element_type=jnp.float32)
             + b2_ref[...])
        g_scr[i] = g
        bmax = jnp.max(g)
        prev = jnp.where(i == 0, -jnp.inf, k_scr[0])
        k_scr[0] = jnp.maximum(prev, bmax)

    @pl.when(p == 1)
    def _phase1():
        @pl.when(i == 0)
        def _init():
            out_ref[...] = jnp.zeros_like(out_ref)
            se_scr[...] = jnp.zeros_like(se_scr)

        g = g_scr[i]
        e = jnp.exp(g - k_scr[0])
        seg = batch_ref[0]
        onehot = (lax.broadcasted_iota(jnp.int32, (NSEG, BLK), 0)
                  == seg).astype(jnp.float32)
        se_scr[...] += jnp.dot(onehot, e, preferred_element_type=jnp.float32)
        xb = x_ref[...]
        for hh in range(NHEADS):
            ex = e[:, hh:hh + 1] * xb
            out_ref[:, hh * D:(hh + 1) * D] += jnp.dot(
                onehot, ex, preferred_element_type=jnp.float32)

    @pl.when((p == 1) & (i == nblk - 1))
    def _finish():
        se = se_scr[...]
        for hh in range(NHEADS):
            out_ref[:, hh * D:(hh + 1) * D] = (
                out_ref[:, hh * D:(hh + 1) * D]
                / (se[:, hh:hh + 1] + 1e-16))


@functools.partial(jax.jit, static_argnames=())
def kernel(x, batch, W1_0, b1_0, W2_0, b2_0, W1_1, b1_1, W2_1, b2_1,
           W1_2, b1_2, W2_2, b2_2, W1_3, b1_3, W2_3, b2_3):
    n = x.shape[0]
    nblk = -(-n // BLK)
    npad = nblk * BLK
    x_p = jnp.pad(x, ((0, npad - n), (0, 0)))
    batch_p = jnp.pad(batch, (0, npad - n), constant_values=NSEG)
    batch_p = batch_p.reshape(nblk, 1, BLK)

    w1t = jnp.concatenate([W1_0.T, W1_1.T, W1_2.T, W1_3.T], axis=1)
    b1c = jnp.concatenate([b1_0, b1_1, b1_2, b1_3]).reshape(1, 4 * 128)
    w2blk = jnp.zeros((4 * 128, NHEADS), jnp.float32)
    for hh, w2 in enumerate([W2_0, W2_1, W2_2, W2_3]):
        w2blk = w2blk.at[hh * 128:(hh + 1) * 128, hh].set(w2[0])
    b2c = jnp.stack([b2_0[0], b2_1[0], b2_2[0], b2_3[0]]).reshape(1, NHEADS)

    out = pl.pallas_call(
        _fused_body,
        grid=(2, nblk),
        in_specs=[
            pl.BlockSpec((1, 1, BLK), lambda p, i: (i, 0, 0)),
            pl.BlockSpec((BLK, D), lambda p, i: (i, 0)),
            pl.BlockSpec((D, 4 * 128), lambda p, i: (0, 0)),
            pl.BlockSpec((1, 4 * 128), lambda p, i: (0, 0)),
            pl.BlockSpec((4 * 128, NHEADS), lambda p, i: (0, 0)),
            pl.BlockSpec((1, NHEADS), lambda p, i: (0, 0)),
        ],
        out_specs=pl.BlockSpec((NSEG, NHEADS * D), lambda p, i: (0, 0)),
        out_shape=jax.ShapeDtypeStruct((NSEG, NHEADS * D), jnp.float32),
        scratch_shapes=[
            pltpu.VMEM((nblk, BLK, NHEADS), jnp.float32),
            pltpu.VMEM((NSEG, NHEADS), jnp.float32),
            pltpu.SMEM((1,), jnp.float32),
        ],
        compiler_params=pltpu.CompilerParams(
            dimension_semantics=("arbitrary", "arbitrary")),
    )(batch_p, x_p, w1t, b1c, w2blk, b2c)
    return out


# speedup vs baseline: 23.4290x; 1.1179x over previous
import functools

import jax
import jax.numpy as jnp
from jax import lax
from jax.experimental import pallas as pl
from jax.experimental.pallas import tpu as pltpu

NSEG = 512
D = 128
NHEADS = 4
BLK = 1024
LSPAN = 256


def _fused_body(batch_ref, x_ref, w1_ref, b1_ref, w2_ref, b2_ref,
                out_ref, g_scr, se_scr, k_scr):
    p = pl.program_id(0)
    i = pl.program_id(1)
    nblk = pl.num_programs(1)

    @pl.when(p == 0)
    def _phase0():
        xb = x_ref[...]
        h = jnp.maximum(
            jnp.dot(xb, w1_ref[...], preferred_element_type=jnp.float32)
            + b1_ref[...], 0.0)
        g = (jnp.dot(h, w2_ref[...], preferred_element_type=jnp.float32)
             + b2_ref[...])
        g_scr[i] = g
        bmax = jnp.max(g)
        prev = jnp.where(i == 0, -jnp.inf, k_scr[0])
        k_scr[0] = jnp.maximum(prev, bmax)

    @pl.when(p == 1)
    def _phase1():
        @pl.when(i == 0)
        def _init():
            out_ref[...] = jnp.zeros_like(out_ref)
            se_scr[...] = jnp.zeros_like(se_scr)

        g = g_scr[i]
        e = jnp.exp(g - k_scr[0])
        seg = batch_ref[0]
        xb = x_ref[...]
        ex = jnp.concatenate(
            [e[:, hh:hh + 1] * xb for hh in range(NHEADS)], axis=1)

        s0 = seg[0, 0]
        s0a = jnp.minimum((s0 // 8) * 8, NSEG - LSPAN)
        smax = seg[0, BLK - 1]
        fits = (smax - s0a) < LSPAN

        @pl.when(fits)
        def _local():
            oh = (lax.broadcasted_iota(jnp.int32, (LSPAN, BLK), 0) + s0a
                  == seg).astype(jnp.float32)
            se_scr[pl.ds(s0a, LSPAN), :] += jnp.dot(
                oh, e, preferred_element_type=jnp.float32)
            out_ref[pl.ds(s0a, LSPAN), :] += jnp.dot(
                oh, ex, preferred_element_type=jnp.float32)

        @pl.when(jnp.logical_not(fits))
        def _full():
            oh = (lax.broadcasted_iota(jnp.int32, (NSEG, BLK), 0)
                  == seg).astype(jnp.float32)
            se_scr[...] += jnp.dot(oh, e, preferred_element_type=jnp.float32)
            out_ref[...] += jnp.dot(oh, ex, preferred_element_type=jnp.float32)

    @pl.when((p == 1) & (i == nblk - 1))
    def _finish():
        se = se_scr[...]
        for hh in range(NHEADS):
            out_ref[:, hh * D:(hh + 1) * D] = (
                out_ref[:, hh * D:(hh + 1) * D]
                / (se[:, hh:hh + 1] + 1e-16))


@functools.partial(jax.jit, static_argnames=())
def kernel(x, batch, W1_0, b1_0, W2_0, b2_0, W1_1, b1_1, W2_1, b2_1,
           W1_2, b1_2, W2_2, b2_2, W1_3, b1_3, W2_3, b2_3):
    n = x.shape[0]
    nblk = -(-n // BLK)
    npad = nblk * BLK
    x_p = jnp.pad(x, ((0, npad - n), (0, 0)))
    batch_p = jnp.pad(batch, (0, npad - n), constant_values=NSEG)
    batch_p = batch_p.reshape(nblk, 1, BLK)

    w1t = jnp.concatenate([W1_0.T, W1_1.T, W1_2.T, W1_3.T], axis=1)
    b1c = jnp.concatenate([b1_0, b1_1, b1_2, b1_3]).reshape(1, 4 * 128)
    w2blk = jnp.zeros((4 * 128, NHEADS), jnp.float32)
    for hh, w2 in enumerate([W2_0, W2_1, W2_2, W2_3]):
        w2blk = w2blk.at[hh * 128:(hh + 1) * 128, hh].set(w2[0])
    b2c = jnp.stack([b2_0[0], b2_1[0], b2_2[0], b2_3[0]]).reshape(1, NHEADS)

    out = pl.pallas_call(
        _fused_body,
        grid=(2, nblk),
        in_specs=[
            pl.BlockSpec((1, 1, BLK), lambda p, i: (i, 0, 0)),
            pl.BlockSpec((BLK, D), lambda p, i: (i, 0)),
            pl.BlockSpec((D, 4 * 128), lambda p, i: (0, 0)),
            pl.BlockSpec((1, 4 * 128), lambda p, i: (0, 0)),
            pl.BlockSpec((4 * 128, NHEADS), lambda p, i: (0, 0)),
            pl.BlockSpec((1, NHEADS), lambda p, i: (0, 0)),
        ],
        out_specs=pl.BlockSpec((NSEG, NHEADS * D), lambda p, i: (0, 0)),
        out_shape=jax.ShapeDtypeStruct((NSEG, NHEADS * D), jnp.float32),
        scratch_shapes=[
            pltpu.VMEM((nblk, BLK, NHEADS), jnp.float32),
            pltpu.VMEM((NSEG, NHEADS), jnp.float32),
            pltpu.SMEM((1,), jnp.float32),
        ],
        compiler_params=pltpu.CompilerParams(
            dimension_semantics=("arbitrary", "arbitrary")),
    )(batch_p, x_p, w1t, b1c, w2blk, b2c)
    return out


# speedup vs baseline: 25.3694x; 1.0828x over previous
import functools

import jax
import jax.numpy as jnp
from jax import lax
from jax.experimental import pallas as pl
from jax.experimental.pallas import tpu as pltpu

NSEG = 512
D = 128
NHEADS = 4
BLK = 1024
LSPAN = 64


def _fused_body(batch_ref, x_ref, w1_ref, b1_ref, w2_ref, b2_ref,
                out_ref, g_scr, se_scr, k_scr):
    p = pl.program_id(0)
    i = pl.program_id(1)
    nblk = pl.num_programs(1)

    @pl.when(p == 0)
    def _phase0():
        xb = x_ref[...]
        h = jnp.maximum(
            jnp.dot(xb, w1_ref[...], preferred_element_type=jnp.float32)
            + b1_ref[...], 0.0)
        g = (jnp.dot(h, w2_ref[...], preferred_element_type=jnp.float32)
             + b2_ref[...])
        g_scr[i] = g
        bmax = jnp.max(g)
        prev = jnp.where(i == 0, -jnp.inf, k_scr[0])
        k_scr[0] = jnp.maximum(prev, bmax)

    @pl.when(p == 1)
    def _phase1():
        @pl.when(i == 0)
        def _init():
            out_ref[...] = jnp.zeros_like(out_ref)
            se_scr[...] = jnp.zeros_like(se_scr)

        g = g_scr[i]
        e = jnp.exp(g - k_scr[0])
        seg = batch_ref[0]
        xb = x_ref[...]
        ex = jnp.concatenate(
            [e[:, hh:hh + 1] * xb for hh in range(NHEADS)], axis=1)

        s0 = seg[0, 0]
        s0a = jnp.minimum((s0 // 8) * 8, NSEG - LSPAN)
        smax = seg[0, BLK - 1]
        fits = (smax - s0a) < LSPAN

        @pl.when(fits)
        def _local():
            oh = (lax.broadcasted_iota(jnp.int32, (LSPAN, BLK), 0) + s0a
                  == seg).astype(jnp.float32)
            se_scr[pl.ds(s0a, LSPAN), :] += jnp.dot(
                oh, e, preferred_element_type=jnp.float32)
            out_ref[pl.ds(s0a, LSPAN), :] += jnp.dot(
                oh, ex, preferred_element_type=jnp.float32)

        @pl.when(jnp.logical_not(fits))
        def _full():
            oh = (lax.broadcasted_iota(jnp.int32, (NSEG, BLK), 0)
                  == seg).astype(jnp.float32)
            se_scr[...] += jnp.dot(oh, e, preferred_element_type=jnp.float32)
            out_ref[...] += jnp.dot(oh, ex, preferred_element_type=jnp.float32)

    @pl.when((p == 1) & (i == nblk - 1))
    def _finish():
        se = se_scr[...]
        for hh in range(NHEADS):
            out_ref[:, hh * D:(hh + 1) * D] = (
                out_ref[:, hh * D:(hh + 1) * D]
                / (se[:, hh:hh + 1] + 1e-16))


@functools.partial(jax.jit, static_argnames=())
def kernel(x, batch, W1_0, b1_0, W2_0, b2_0, W1_1, b1_1, W2_1, b2_1,
           W1_2, b1_2, W2_2, b2_2, W1_3, b1_3, W2_3, b2_3):
    n = x.shape[0]
    nblk = -(-n // BLK)
    npad = nblk * BLK
    x_p = jnp.pad(x, ((0, npad - n), (0, 0)))
    batch_p = jnp.pad(batch, (0, npad - n), constant_values=NSEG)
    batch_p = batch_p.reshape(nblk, 1, BLK)

    w1t = jnp.concatenate([W1_0.T, W1_1.T, W1_2.T, W1_3.T], axis=1)
    b1c = jnp.concatenate([b1_0, b1_1, b1_2, b1_3]).reshape(1, 4 * 128)
    w2blk = jnp.zeros((4 * 128, NHEADS), jnp.float32)
    for hh, w2 in enumerate([W2_0, W2_1, W2_2, W2_3]):
        w2blk = w2blk.at[hh * 128:(hh + 1) * 128, hh].set(w2[0])
    b2c = jnp.stack([b2_0[0], b2_1[0], b2_2[0], b2_3[0]]).reshape(1, NHEADS)

    out = pl.pallas_call(
        _fused_body,
        grid=(2, nblk),
        in_specs=[
            pl.BlockSpec((1, 1, BLK), lambda p, i: (i, 0, 0)),
            pl.BlockSpec((BLK, D), lambda p, i: (i, 0)),
            pl.BlockSpec((D, 4 * 128), lambda p, i: (0, 0)),
            pl.BlockSpec((1, 4 * 128), lambda p, i: (0, 0)),
            pl.BlockSpec((4 * 128, NHEADS), lambda p, i: (0, 0)),
            pl.BlockSpec((1, NHEADS), lambda p, i: (0, 0)),
        ],
        out_specs=pl.BlockSpec((NSEG, NHEADS * D), lambda p, i: (0, 0)),
        out_shape=jax.ShapeDtypeStruct((NSEG, NHEADS * D), jnp.float32),
        scratch_shapes=[
            pltpu.VMEM((nblk, BLK, NHEADS), jnp.float32),
            pltpu.VMEM((NSEG, NHEADS), jnp.float32),
            pltpu.SMEM((1,), jnp.float32),
        ],
        compiler_params=pltpu.CompilerParams(
            dimension_semantics=("arbitrary", "arbitrary")),
    )(batch_p, x_p, w1t, b1c, w2blk, b2c)
    return out


# speedup vs baseline: 34.3444x; 1.3538x over previous
import functools

import jax
import jax.numpy as jnp
from jax import lax
from jax.experimental import pallas as pl
from jax.experimental.pallas import tpu as pltpu

NSEG = 512
D = 128
NHEADS = 4
BLK = 2048
LSPAN = 64


def _fused_body(batch_ref, x_ref, w1_ref, b1_ref, w2_ref, b2_ref,
                out_ref, g_scr, se_scr, k_scr):
    p = pl.program_id(0)
    i = pl.program_id(1)
    nblk = pl.num_programs(1)

    @pl.when(p == 0)
    def _phase0():
        xb = x_ref[...]
        h = jnp.maximum(
            jnp.dot(xb, w1_ref[...], preferred_element_type=jnp.float32)
            + b1_ref[...], 0.0)
        g = (jnp.dot(h, w2_ref[...], preferred_element_type=jnp.float32)
             + b2_ref[...])
        g_scr[i] = g
        bmax = jnp.max(g)
        prev = jnp.where(i == 0, -jnp.inf, k_scr[0])
        k_scr[0] = jnp.maximum(prev, bmax)

    @pl.when(p == 1)
    def _phase1():
        @pl.when(i == 0)
        def _init():
            out_ref[...] = jnp.zeros_like(out_ref)
            se_scr[...] = jnp.zeros_like(se_scr)

        g = g_scr[i]
        e = jnp.exp(g - k_scr[0]).astype(jnp.bfloat16)
        seg = batch_ref[0]
        xb = x_ref[...]
        ex = jnp.concatenate(
            [e[:, hh:hh + 1] * xb for hh in range(NHEADS)], axis=1)

        s0 = seg[0, 0]
        s0a = jnp.minimum((s0 // 8) * 8, NSEG - LSPAN)
        smax = seg[0, BLK - 1]
        fits = (smax - s0a) < LSPAN

        @pl.when(fits)
        def _local():
            oh = (lax.broadcasted_iota(jnp.int32, (LSPAN, BLK), 0) + s0a
                  == seg).astype(jnp.bfloat16)
            se_scr[pl.ds(s0a, LSPAN), :] += jnp.dot(
                oh, e, preferred_element_type=jnp.float32)
            out_ref[pl.ds(s0a, LSPAN), :] += jnp.dot(
                oh, ex, preferred_element_type=jnp.float32)

        @pl.when(jnp.logical_not(fits))
        def _full():
            oh = (lax.broadcasted_iota(jnp.int32, (NSEG, BLK), 0)
                  == seg).astype(jnp.bfloat16)
            se_scr[...] += jnp.dot(oh, e, preferred_element_type=jnp.float32)
            out_ref[...] += jnp.dot(oh, ex, preferred_element_type=jnp.float32)

    @pl.when((p == 1) & (i == nblk - 1))
    def _finish():
        se = se_scr[...]
        for hh in range(NHEADS):
            out_ref[:, hh * D:(hh + 1) * D] = (
                out_ref[:, hh * D:(hh + 1) * D]
                / (se[:, hh:hh + 1] + 1e-16))


@functools.partial(jax.jit, static_argnames=())
def kernel(x, batch, W1_0, b1_0, W2_0, b2_0, W1_1, b1_1, W2_1, b2_1,
           W1_2, b1_2, W2_2, b2_2, W1_3, b1_3, W2_3, b2_3):
    n = x.shape[0]
    nblk = -(-n // BLK)
    npad = nblk * BLK
    x_p = jnp.pad(x, ((0, npad - n), (0, 0))).astype(jnp.bfloat16)
    batch_p = jnp.pad(batch, (0, npad - n), constant_values=NSEG)
    batch_p = batch_p.reshape(nblk, 1, BLK)

    w1t = jnp.concatenate([W1_0.T, W1_1.T, W1_2.T, W1_3.T], axis=1).astype(jnp.bfloat16)
    b1c = jnp.concatenate([b1_0, b1_1, b1_2, b1_3]).reshape(1, 4 * 128)
    w2blk = jnp.zeros((4 * 128, NHEADS), jnp.float32)
    for hh, w2 in enumerate([W2_0, W2_1, W2_2, W2_3]):
        w2blk = w2blk.at[hh * 128:(hh + 1) * 128, hh].set(w2[0])
    b2c = jnp.stack([b2_0[0], b2_1[0], b2_2[0], b2_3[0]]).reshape(1, NHEADS)

    out = pl.pallas_call(
        _fused_body,
        grid=(2, nblk),
        in_specs=[
            pl.BlockSpec((1, 1, BLK), lambda p, i: (i, 0, 0)),
            pl.BlockSpec((BLK, D), lambda p, i: (i, 0)),
            pl.BlockSpec((D, 4 * 128), lambda p, i: (0, 0)),
            pl.BlockSpec((1, 4 * 128), lambda p, i: (0, 0)),
            pl.BlockSpec((4 * 128, NHEADS), lambda p, i: (0, 0)),
            pl.BlockSpec((1, NHEADS), lambda p, i: (0, 0)),
        ],
        out_specs=pl.BlockSpec((NSEG, NHEADS * D), lambda p, i: (0, 0)),
        out_shape=jax.ShapeDtypeStruct((NSEG, NHEADS * D), jnp.float32),
        scratch_shapes=[
            pltpu.VMEM((nblk, BLK, NHEADS), jnp.float32),
            pltpu.VMEM((NSEG, NHEADS), jnp.float32),
            pltpu.SMEM((1,), jnp.float32),
        ],
        compiler_params=pltpu.CompilerParams(
            dimension_semantics=("arbitrary", "arbitrary")),
    )(batch_p, x_p, w1t, b1c, w2blk, b2c)
    return out
